# Initial kernel scaffold; baseline (speedup 1.0000x reference)
import jax
import jax.numpy as jnp
from jax.experimental import pallas as pl


def kernel(ids, mask, edge_index, node_features, lm_embed, W_mlp, b_mlp, W1, b1, W2, b2):
    raise NotImplementedError("write your pallas kernel here")


# speedup vs baseline: 12.8732x; 12.8732x over previous
import functools

import jax
import jax.numpy as jnp
from jax import lax
from jax.experimental import pallas as pl
from jax.experimental.pallas import tpu as pltpu
from jax.experimental.pallas import tpu_sc as plsc

N_CORES = 2
N_SUB = 16
N_TILES = N_CORES * N_SUB

N_NODES = 10000
NP = 10240
N_EDGES = 320000
D = 128
DH = D // 2
B = 8
LM_DIM = 1024

EPT = N_EDGES // N_SUB
CHUNK = 80
NCHUNK = EPT // CHUNK
HCHUNK = NCHUNK // N_CORES
ROWS_PER_TILE = NP // N_SUB
SLAB = 128

_MESH = plsc.VectorSubcoreMesh(core_axis_name="c", subcore_axis_name="s")


def _deg_body(dst_hbm, ids_hbm, emb_hbm, deg_hbm, cls_hbm,
              dst_v, ones_v, zbuf_v, ids_v, row_v, dega_sp, sem):
    c = lax.axis_index("c")
    s = lax.axis_index("s")
    wid = s * N_CORES + c
    pltpu.sync_copy(dst_hbm.at[s, pl.ds(c * HCHUNK, HCHUNK), :], dst_v)
    zero16 = jnp.zeros((16,), jnp.float32)
    ones16 = jnp.ones((16,), jnp.float32)

    def _fill(i, carry):
        zbuf_v[i, :] = zero16
        return carry

    lax.fori_loop(0, SLAB, _fill, 0)

    def _fill1(i, carry):
        ones_v[i, :] = ones16
        return carry

    lax.fori_loop(0, CHUNK, _fill1, 0)
    for k in range(ROWS_PER_TILE // SLAB):
        pltpu.sync_copy(
            zbuf_v, dega_sp.at[pl.ds(s * ROWS_PER_TILE + k * SLAB, SLAB), :])
    plsc.subcore_barrier()

    def _edge_chunk(j, carry):
        pltpu.sync_copy(ones_v, dega_sp.at[dst_v.at[j]], add=True)
        return carry

    lax.fori_loop(0, HCHUNK, _edge_chunk, 0)
    plsc.subcore_barrier()
    sl = pl.ds(s * ROWS_PER_TILE, ROWS_PER_TILE)
    pltpu.sync_copy(dega_sp.at[sl, :], deg_hbm.at[c, sl, :])

    @pl.when(wid == 0)
    def _gather_cls():
        pltpu.sync_copy(ids_hbm, ids_v)
        pltpu.async_copy(emb_hbm.at[ids_v], row_v, sem).wait()
        pltpu.sync_copy(row_v, cls_hbm)


_deg_kernel = functools.partial(
    pl.kernel,
    out_type=[
        jax.ShapeDtypeStruct((N_CORES, NP, 16), jnp.float32),
        jax.ShapeDtypeStruct((B, LM_DIM), jnp.float32),
    ],
    mesh=_MESH,
    compiler_params=pltpu.CompilerParams(use_tc_tiling_on_sc=False),
    scratch_types=[
        pltpu.VMEM((HCHUNK, CHUNK), jnp.int32),
        pltpu.VMEM((CHUNK, 16), jnp.float32),
        pltpu.VMEM((SLAB, 16), jnp.float32),
        pltpu.VMEM((B,), jnp.int32),
        pltpu.VMEM((B, LM_DIM), jnp.float32),
        pltpu.VMEM_SHARED((NP, 16), jnp.float32),
        pltpu.SemaphoreType.DMA,
    ],
)(_deg_body)


def _scatter_body(ysl_hbm, ysr_hbm, src_hbm, dst_hbm, out_hbm,
                  src_v, dst_v, rows_v, zbuf_v, acc_sp, sem):
    c = lax.axis_index("c")
    s = lax.axis_index("s")
    pltpu.sync_copy(src_hbm.at[s], src_v)
    pltpu.sync_copy(dst_hbm.at[s], dst_v)
    zero16 = jnp.zeros((16,), jnp.float32)

    def _zero(i, carry):
        zbuf_v[i // 4, pl.ds((i % 4) * 16, 16)] = zero16
        return carry

    lax.fori_loop(0, SLAB * 4, _zero, 0)
    for k in range(ROWS_PER_TILE // SLAB):
        pltpu.sync_copy(
            zbuf_v, acc_sp.at[pl.ds(s * ROWS_PER_TILE + k * SLAB, SLAB), :])
    plsc.subcore_barrier()

    def _edge_chunk(j, carry):
        idx = src_v.at[j]

        @pl.when(c == 0)
        def _gl():
            pltpu.async_copy(ysl_hbm.at[idx], rows_v, sem).wait()

        @pl.when(c == 1)
        def _gr():
            pltpu.async_copy(ysr_hbm.at[idx], rows_v, sem).wait()

        pltpu.sync_copy(rows_v, acc_sp.at[dst_v.at[j]], add=True)
        return carry

    lax.fori_loop(0, NCHUNK, _edge_chunk, 0)
    plsc.subcore_barrier()
    for k in range(ROWS_PER_TILE // SLAB):
        sl = pl.ds(s * ROWS_PER_TILE + k * SLAB, SLAB)
        pltpu.sync_copy(acc_sp.at[sl, :], out_hbm.at[c, sl, :])


_scatter_kernel = functools.partial(
    pl.kernel,
    out_type=jax.ShapeDtypeStruct((N_CORES, NP, DH), jnp.float32),
    mesh=_MESH,
    compiler_params=pltpu.CompilerParams(use_tc_tiling_on_sc=False),
    scratch_types=[
        pltpu.VMEM((NCHUNK, CHUNK), jnp.int32),
        pltpu.VMEM((NCHUNK, CHUNK), jnp.int32),
        pltpu.VMEM((CHUNK, DH), jnp.float32),
        pltpu.VMEM((SLAB, DH), jnp.float32),
        pltpu.VMEM_SHARED((NP, DH), jnp.float32),
        pltpu.SemaphoreType.DMA,
    ],
)(_scatter_body)


_BN = 1024


def _tc1_body(x_ref, w_ref, degp_ref, ysl_ref, ysr_ref, dinv_ref):
    deg = 1.0 + jnp.sum(degp_ref[...], axis=(0, 2)) * (1.0 / 16.0)
    dinv = lax.rsqrt(deg)
    y = jnp.dot(x_ref[...], w_ref[...], preferred_element_type=jnp.float32)
    ys = y * dinv[:, None]
    ysl_ref[...] = ys[:, :DH]
    ysr_ref[...] = ys[:, DH:]
    dinv_ref[...] = dinv[:, None]


def _tc1(x, w1, deg_partials):
    return pl.pallas_call(
        _tc1_body,
        grid=(NP // _BN,),
        in_specs=[
            pl.BlockSpec((_BN, D), lambda i: (i, 0)),
            pl.BlockSpec((D, D), lambda i: (0, 0)),
            pl.BlockSpec((N_CORES, _BN, 16), lambda i: (0, i, 0)),
        ],
        out_specs=[
            pl.BlockSpec((_BN, DH), lambda i: (i, 0)),
            pl.BlockSpec((_BN, DH), lambda i: (i, 0)),
            pl.BlockSpec((_BN, 1), lambda i: (i, 0)),
        ],
        out_shape=[
            jax.ShapeDtypeStruct((NP, DH), jnp.float32),
            jax.ShapeDtypeStruct((NP, DH), jnp.float32),
            jax.ShapeDtypeStruct((NP, 1), jnp.float32),
        ],
    )(x, w1, deg_partials)


def _tc2_body(acc_ref, ysl_ref, ysr_ref, dinv_ref, b_ref, w_ref,
              ys2l_ref, ys2r_ref):
    dinv = dinv_ref[...]
    agg = jnp.concatenate([acc_ref[0] + ysl_ref[...],
                           acc_ref[1] + ysr_ref[...]], axis=-1)
    tot = agg * dinv + b_ref[...]
    h = jnp.maximum(tot, 0.0)
    y2 = jnp.dot(h, w_ref[...], preferred_element_type=jnp.float32)
    ys2 = y2 * dinv
    ys2l_ref[...] = ys2[:, :DH]
    ys2r_ref[...] = ys2[:, DH:]


def _tc2(acc1, ysl, ysr, dinv, b1, w2):
    return pl.pallas_call(
        _tc2_body,
        grid=(NP // _BN,),
        in_specs=[
            pl.BlockSpec((N_CORES, _BN, DH), lambda i: (0, i, 0)),
            pl.BlockSpec((_BN, DH), lambda i: (i, 0)),
            pl.BlockSpec((_BN, DH), lambda i: (i, 0)),
            pl.BlockSpec((_BN, 1), lambda i: (i, 0)),
            pl.BlockSpec((1, D), lambda i: (0, 0)),
            pl.BlockSpec((D, D), lambda i: (0, 0)),
        ],
        out_specs=[
            pl.BlockSpec((_BN, DH), lambda i: (i, 0)),
            pl.BlockSpec((_BN, DH), lambda i: (i, 0)),
        ],
        out_shape=[
            jax.ShapeDtypeStruct((NP, DH), jnp.float32),
            jax.ShapeDtypeStruct((NP, DH), jnp.float32),
        ],
    )(acc1, ysl, ysr, dinv, b1, w2)


def _tc3_body(acc_ref, ysl_ref, ysr_ref, dinv_ref, b_ref, out_ref):
    agg = jnp.concatenate([acc_ref[0] + ysl_ref[...],
                           acc_ref[1] + ysr_ref[...]], axis=-1)
    out_ref[...] = agg * dinv_ref[...] + b_ref[...]


def _tc3(acc2, ysl, ysr, dinv, b2):
    return pl.pallas_call(
        _tc3_body,
        grid=(NP // _BN,),
        in_specs=[
            pl.BlockSpec((N_CORES, _BN, DH), lambda i: (0, i, 0)),
            pl.BlockSpec((_BN, DH), lambda i: (i, 0)),
            pl.BlockSpec((_BN, DH), lambda i: (i, 0)),
            pl.BlockSpec((_BN, 1), lambda i: (i, 0)),
            pl.BlockSpec((1, D), lambda i: (0, 0)),
        ],
        out_specs=pl.BlockSpec((_BN, D), lambda i: (i, 0)),
        out_shape=jax.ShapeDtypeStruct((NP, D), jnp.float32),
    )(acc2, ysl, ysr, dinv, b2)


def _lm_body(cls_ref, mask_ref, w_ref, b_ref, out_ref):
    row = cls_ref[...] * mask_ref[...]
    out_ref[...] = (jnp.dot(row, w_ref[...], preferred_element_type=jnp.float32)
                    + b_ref[...])


def _lm(cls_rows, mask_col, w_mlp, b_mlp):
    mlp_out = w_mlp.shape[1]
    return pl.pallas_call(
        _lm_body,
        out_shape=jax.ShapeDtypeStruct((B, mlp_out), jnp.float32),
    )(cls_rows, mask_col, w_mlp, b_mlp)


def kernel(ids, mask, edge_index, node_features, lm_embed,
           W_mlp, b_mlp, W1, b1, W2, b2):
    src_r = edge_index[0].reshape(N_SUB, NCHUNK, CHUNK)
    dst_r = edge_index[1].reshape(N_SUB, NCHUNK, CHUNK)
    x = jnp.pad(node_features, ((0, NP - N_NODES), (0, 0)))

    deg_partials, cls_rows = _deg_kernel(dst_r, ids[:, 0], lm_embed)
    ys1l, ys1r, dinv = _tc1(x, W1, deg_partials)
    acc1 = _scatter_kernel(ys1l, ys1r, src_r, dst_r)
    ys2l, ys2r = _tc2(acc1, ys1l, ys1r, dinv, b1.reshape(1, D), W2)
    acc2 = _scatter_kernel(ys2l, ys2r, src_r, dst_r)
    gcn_out = _tc3(acc2, ys2l, ys2r, dinv, b2.reshape(1, D))

    lm_embeddings = _lm(cls_rows, mask[:, :1].astype(jnp.float32),
                        W_mlp, b_mlp.reshape(1, -1))
    return (lm_embeddings, gcn_out[:N_NODES])
